# baseline (speedup 1.0000x reference)
import functools

import jax
import jax.numpy as jnp
from jax import lax
from jax.experimental import pallas as pl
from jax.experimental.pallas import tpu as pltpu
from jax.experimental.pallas import tpu_sc as plsc

N = 10000
E = 320000
D = 128
NC, NS = 2, 16
NW = NC * NS
CH = 128
G = 4
FAST = 0
KCF = 80
KCS = 80
NGF = KCF // G
NGS = KCS // G
E_PAD = NS * (KCF + KCS) * CH
RPS = 640
N_PAD = NS * RPS


@functools.cache
def _make_agg(with_deg):
    outs = [jax.ShapeDtypeStruct((NC * N_PAD, D), jnp.float32)]
    scratch = [
        pltpu.VMEM((G, CH), jnp.int32),
        pltpu.VMEM((G, CH), jnp.int32),
        pltpu.VMEM((CH, D), jnp.float32),
        pltpu.VMEM((CH, D), jnp.float32),
        pltpu.VMEM_SHARED((N_PAD, D), jnp.float32),
    ] + [pltpu.SemaphoreType.DMA] * 6
    if with_deg:
        outs.append(jax.ShapeDtypeStruct((NW, N_PAD), jnp.float32))
        scratch.append(pltpu.VMEM((N_PAD,), jnp.float32))
    mesh = plsc.VectorSubcoreMesh(core_axis_name="c", subcore_axis_name="s",
                                  num_cores=NC, num_subcores=NS)

    def body(z_hbm, z2_hbm, srcf_hbm, dstf_hbm, srcs_hbm, dsts_hbm,
             zrow_hbm, out_hbm, *rest):
        if with_deg:
            (deg_hbm, src_v, dst_v, rows0, rows1, acc,
             sem0, sem1, sem2, sem3, sem4, sem5, deg_t) = rest
        else:
            (src_v, dst_v, rows0, rows1, acc,
             sem0, sem1, sem2, sem3, sem4, sem5) = rest
        rows = (rows0, rows1)
        gsems = ((sem0, sem1), (sem2, sem3))
        ssems = (sem4, sem5)
        H = CH // 2
        cid = lax.axis_index("c")
        sid = lax.axis_index("s")
        widx = cid * NS + sid
        pltpu.sync_copy(zrow_hbm, acc.at[pl.ds(sid * RPS, RPS)])
        if with_deg:
            @pl.loop(0, N_PAD // 16)
            def _zero(i):
                deg_t[pl.ds(i * 16, 16)] = jnp.zeros((16,), jnp.float32)
        plsc.subcore_barrier()
        ones = jnp.ones((16,), jnp.float32)

        def run_pipeline(src_hbm, dst_hbm, ng, zsrc):
            @pl.loop(0, ng)
            def _group(g):
                pltpu.sync_copy(src_hbm.at[sid, pl.ds(g * G, G)], src_v)
                pltpu.sync_copy(dst_hbm.at[sid, pl.ds(g * G, G)], dst_v)
                def gather(j, b):
                    return [
                        pltpu.async_copy(zsrc.at[src_v.at[j, pl.ds(0, H)]],
                                         rows[b].at[pl.ds(0, H)],
                                         gsems[b][0]),
                        pltpu.async_copy(zsrc.at[src_v.at[j, pl.ds(H, H)]],
                                         rows[b].at[pl.ds(H, H)],
                                         gsems[b][1]),
                    ]

                gd = [None, None]
                sd = [None, None]
                gd[0] = gather(0, 0)
                for j in range(G):
                    b = j % 2
                    nb = 1 - b
                    if j >= 1:
                        sd[nb].wait()
                    if j + 1 < G:
                        gd[nb] = gather(j + 1, nb)
                    for d in gd[b]:
                        d.wait()
                    sd[b] = pltpu.async_copy(rows[b], acc.at[dst_v.at[j]],
                                             ssems[b], add=True)
                    if with_deg:
                        for k in range(CH // 16):
                            idx = dst_v[j, pl.ds(k * 16, 16)]
                            plsc.addupdate_scatter(deg_t, [idx], ones)
                sd[(G - 1) % 2].wait()

        @pl.when(cid == FAST)
        def _fast():
            run_pipeline(srcf_hbm, dstf_hbm, NGF, z_hbm)

        @pl.when(cid != FAST)
        def _slow():
            run_pipeline(srcs_hbm, dsts_hbm, NGS, z2_hbm)

        plsc.subcore_barrier()
        base = cid * N_PAD + sid * RPS
        pltpu.sync_copy(acc.at[pl.ds(sid * RPS, RPS)],
                        out_hbm.at[pl.ds(base, RPS)])
        if with_deg:
            pltpu.sync_copy(deg_t, deg_hbm.at[widx])

    params = (pltpu.CompilerParams(needs_layout_passes=False)
              if with_deg else None)
    return pl.kernel(body, out_type=tuple(outs) if with_deg else outs[0],
                     mesh=mesh, scratch_types=scratch,
                     compiler_params=params)


def _matmul_t(a, w):
    return lax.dot_general(a, w, (((1,), (1,)), ((), ())),
                           preferred_element_type=jnp.float32)


def _dense1_body(x_ref, wl_ref, wr_ref, b_ref, z_ref, z2_ref, r_ref):
    xv = x_ref[...]
    zv = _matmul_t(xv, wl_ref[...])
    z_ref[...] = zv
    z2_ref[...] = zv
    r_ref[...] = _matmul_t(xv, wr_ref[...]) + b_ref[...]


def _mid_body(p0_ref, p1_ref, dg_ref, r1_ref, wl_ref, wr_ref, b_ref,
              z_ref, z2_ref, r_ref):
    deg = jnp.sum(dg_ref[...], axis=1)[:, None]
    inv = 1.0 / jnp.maximum(deg, 1.0)
    h = jnp.maximum((p0_ref[...] + p1_ref[...]) * inv + r1_ref[...], 0.0)
    zv = _matmul_t(h, wl_ref[...])
    z_ref[...] = zv
    z2_ref[...] = zv
    r_ref[...] = _matmul_t(h, wr_ref[...]) + b_ref[...]


def _final_body(p0_ref, p1_ref, dg_ref, r2_ref, o_ref):
    deg = jnp.sum(dg_ref[...], axis=1)[:, None]
    inv = 1.0 / jnp.maximum(deg, 1.0)
    o_ref[...] = (p0_ref[...] + p1_ref[...]) * inv + r2_ref[...]


TB = 1000


def _row_spec(w):
    return pl.BlockSpec((TB, w), lambda i: (i, 0))


def _full_spec(shape):
    return pl.BlockSpec(shape, lambda i: (0,) * len(shape))


_dense1 = pl.pallas_call(
    _dense1_body, grid=(N // TB,),
    in_specs=[_row_spec(D), _full_spec((D, D)), _full_spec((D, D)),
              _full_spec((1, D))],
    out_specs=[_row_spec(D)] * 3,
    out_shape=[jax.ShapeDtypeStruct((N, D), jnp.float32)] * 3,
)

_deg_spec = pl.BlockSpec((TB, NW), lambda i: (i, 0))

_mid = pl.pallas_call(
    _mid_body, grid=(N // TB,),
    in_specs=[_row_spec(D), _row_spec(D), _deg_spec,
              _row_spec(D), _full_spec((D, D)), _full_spec((D, D)),
              _full_spec((1, D))],
    out_specs=[_row_spec(D)] * 3,
    out_shape=[jax.ShapeDtypeStruct((N, D), jnp.float32)] * 3,
)

_final = pl.pallas_call(
    _final_body, grid=(N // TB,),
    in_specs=[_row_spec(D), _row_spec(D), _deg_spec, _row_spec(D)],
    out_specs=_row_spec(D),
    out_shape=jax.ShapeDtypeStruct((N, D), jnp.float32),
)


def kernel(x, edge_index, W_l1, b_l1, W_r1, W_l2, b_l2, W_r2):
    ei = edge_index.astype(jnp.int32)
    pad = E_PAD - E
    src = jnp.concatenate([ei[0], jnp.zeros((pad,), jnp.int32)])
    dst = jnp.concatenate([ei[1], jnp.full((pad,), N, jnp.int32)])
    EF = NS * KCF * CH
    srcf = src[:EF].reshape(NS, KCF, CH)
    dstf = dst[:EF].reshape(NS, KCF, CH)
    srcs = src[EF:].reshape(NS, KCS, CH)
    dsts = dst[EF:].reshape(NS, KCS, CH)
    zrow = jnp.zeros((RPS, D), jnp.float32)

    z1, z1b, r1 = _dense1(x, W_l1, W_r1, b_l1.reshape(1, D))
    part1, degp = _make_agg(True)(z1, z1b, srcf, dstf, srcs, dsts, zrow)
    p0, p1 = part1[:N], part1[N_PAD:N_PAD + N]
    dg = degp[:, :N].T
    z2, z2b, r2 = _mid(p0, p1, dg, r1, W_l2, W_r2, b_l2.reshape(1, D))
    part2 = _make_agg(False)(z2, z2b, srcf, dstf, srcs, dsts, zrow)
    q0, q1 = part2[:N], part2[N_PAD:N_PAD + N]
    return _final(q0, q1, dg, r2)


# speedup vs baseline: 1.3635x; 1.3635x over previous
import functools

import jax
import jax.numpy as jnp
from jax import lax
from jax.experimental import pallas as pl
from jax.experimental.pallas import tpu as pltpu
from jax.experimental.pallas import tpu_sc as plsc

N = 10000
E = 320000
D = 128
NC, NS = 2, 16
NW = NC * NS
CH = 128
G = 4
FAST = 0
KCF = 144
KCS = 16
NGF = KCF // G
NGS = KCS // G
E_PAD = NS * (KCF + KCS) * CH
RPS = 640
N_PAD = NS * RPS


@functools.cache
def _make_agg(with_deg):
    outs = [jax.ShapeDtypeStruct((NC * N_PAD, D), jnp.float32)]
    scratch = [
        pltpu.VMEM((G, CH), jnp.int32),
        pltpu.VMEM((G, CH), jnp.int32),
        pltpu.VMEM((CH, D), jnp.float32),
        pltpu.VMEM((CH, D), jnp.float32),
        pltpu.VMEM_SHARED((N_PAD, D), jnp.float32),
    ] + [pltpu.SemaphoreType.DMA] * 6
    if with_deg:
        outs.append(jax.ShapeDtypeStruct((NW, N_PAD), jnp.float32))
        scratch.append(pltpu.VMEM((N_PAD,), jnp.float32))
    mesh = plsc.VectorSubcoreMesh(core_axis_name="c", subcore_axis_name="s",
                                  num_cores=NC, num_subcores=NS)

    def body(z_hbm, srcf_hbm, dstf_hbm, srcs_hbm, dsts_hbm, zrow_hbm,
             out_hbm, *rest):
        if with_deg:
            (deg_hbm, src_v, dst_v, rows0, rows1, acc,
             sem0, sem1, sem2, sem3, sem4, sem5, deg_t) = rest
        else:
            (src_v, dst_v, rows0, rows1, acc,
             sem0, sem1, sem2, sem3, sem4, sem5) = rest
        rows = (rows0, rows1)
        gsems = ((sem0, sem1), (sem2, sem3))
        ssems = (sem4, sem5)
        H = CH // 2
        cid = lax.axis_index("c")
        sid = lax.axis_index("s")
        widx = cid * NS + sid
        pltpu.sync_copy(zrow_hbm, acc.at[pl.ds(sid * RPS, RPS)])
        if with_deg:
            @pl.loop(0, N_PAD // 16)
            def _zero(i):
                deg_t[pl.ds(i * 16, 16)] = jnp.zeros((16,), jnp.float32)
        plsc.subcore_barrier()
        ones = jnp.ones((16,), jnp.float32)

        def run_pipeline(src_hbm, dst_hbm, ng):
            @pl.loop(0, ng)
            def _group(g):
                pltpu.sync_copy(src_hbm.at[sid, pl.ds(g * G, G)], src_v)
                pltpu.sync_copy(dst_hbm.at[sid, pl.ds(g * G, G)], dst_v)
                def gather(j, b):
                    return [
                        pltpu.async_copy(z_hbm.at[src_v.at[j, pl.ds(0, H)]],
                                         rows[b].at[pl.ds(0, H)],
                                         gsems[b][0]),
                        pltpu.async_copy(z_hbm.at[src_v.at[j, pl.ds(H, H)]],
                                         rows[b].at[pl.ds(H, H)],
                                         gsems[b][1]),
                    ]

                gd = [None, None]
                sd = [None, None]
                gd[0] = gather(0, 0)
                for j in range(G):
                    b = j % 2
                    nb = 1 - b
                    if j >= 1:
                        sd[nb].wait()
                    if j + 1 < G:
                        gd[nb] = gather(j + 1, nb)
                    for d in gd[b]:
                        d.wait()
                    sd[b] = pltpu.async_copy(rows[b], acc.at[dst_v.at[j]],
                                             ssems[b], add=True)
                    if with_deg:
                        for k in range(CH // 16):
                            idx = dst_v[j, pl.ds(k * 16, 16)]
                            plsc.addupdate_scatter(deg_t, [idx], ones)
                sd[(G - 1) % 2].wait()

        @pl.when(cid == FAST)
        def _fast():
            run_pipeline(srcf_hbm, dstf_hbm, NGF)

        @pl.when(cid != FAST)
        def _slow():
            run_pipeline(srcs_hbm, dsts_hbm, NGS)

        plsc.subcore_barrier()
        base = cid * N_PAD + sid * RPS
        pltpu.sync_copy(acc.at[pl.ds(sid * RPS, RPS)],
                        out_hbm.at[pl.ds(base, RPS)])
        if with_deg:
            pltpu.sync_copy(deg_t, deg_hbm.at[widx])

    params = (pltpu.CompilerParams(needs_layout_passes=False)
              if with_deg else None)
    return pl.kernel(body, out_type=tuple(outs) if with_deg else outs[0],
                     mesh=mesh, scratch_types=scratch,
                     compiler_params=params)


def _matmul_t(a, w):
    return lax.dot_general(a, w, (((1,), (1,)), ((), ())),
                           preferred_element_type=jnp.float32)


def _dense1_body(x_ref, wl_ref, wr_ref, b_ref, z_ref, r_ref):
    xv = x_ref[...]
    z_ref[...] = _matmul_t(xv, wl_ref[...])
    r_ref[...] = _matmul_t(xv, wr_ref[...]) + b_ref[...]


def _mid_body(p0_ref, p1_ref, dg_ref, r1_ref, wl_ref, wr_ref, b_ref,
              z_ref, r_ref):
    deg = jnp.sum(dg_ref[...], axis=1)[:, None]
    inv = 1.0 / jnp.maximum(deg, 1.0)
    h = jnp.maximum((p0_ref[...] + p1_ref[...]) * inv + r1_ref[...], 0.0)
    z_ref[...] = _matmul_t(h, wl_ref[...])
    r_ref[...] = _matmul_t(h, wr_ref[...]) + b_ref[...]


def _final_body(p0_ref, p1_ref, dg_ref, r2_ref, o_ref):
    deg = jnp.sum(dg_ref[...], axis=1)[:, None]
    inv = 1.0 / jnp.maximum(deg, 1.0)
    o_ref[...] = (p0_ref[...] + p1_ref[...]) * inv + r2_ref[...]


TB = 1000


def _row_spec(w):
    return pl.BlockSpec((TB, w), lambda i: (i, 0))


def _full_spec(shape):
    return pl.BlockSpec(shape, lambda i: (0,) * len(shape))


_dense1 = pl.pallas_call(
    _dense1_body, grid=(N // TB,),
    in_specs=[_row_spec(D), _full_spec((D, D)), _full_spec((D, D)),
              _full_spec((1, D))],
    out_specs=[_row_spec(D), _row_spec(D)],
    out_shape=[jax.ShapeDtypeStruct((N, D), jnp.float32)] * 2,
)

_deg_spec = pl.BlockSpec((TB, NW), lambda i: (i, 0))

_mid = pl.pallas_call(
    _mid_body, grid=(N // TB,),
    in_specs=[_row_spec(D), _row_spec(D), _deg_spec,
              _row_spec(D), _full_spec((D, D)), _full_spec((D, D)),
              _full_spec((1, D))],
    out_specs=[_row_spec(D), _row_spec(D)],
    out_shape=[jax.ShapeDtypeStruct((N, D), jnp.float32)] * 2,
)

_final = pl.pallas_call(
    _final_body, grid=(N // TB,),
    in_specs=[_row_spec(D), _row_spec(D), _deg_spec, _row_spec(D)],
    out_specs=_row_spec(D),
    out_shape=jax.ShapeDtypeStruct((N, D), jnp.float32),
)


def kernel(x, edge_index, W_l1, b_l1, W_r1, W_l2, b_l2, W_r2):
    ei = edge_index.astype(jnp.int32)
    pad = E_PAD - E
    src = jnp.concatenate([ei[0], jnp.zeros((pad,), jnp.int32)])
    dst = jnp.concatenate([ei[1], jnp.full((pad,), N, jnp.int32)])
    EF = NS * KCF * CH
    srcf = src[:EF].reshape(NS, KCF, CH)
    dstf = dst[:EF].reshape(NS, KCF, CH)
    srcs = src[EF:].reshape(NS, KCS, CH)
    dsts = dst[EF:].reshape(NS, KCS, CH)
    zrow = jnp.zeros((RPS, D), jnp.float32)

    z1, r1 = _dense1(x, W_l1, W_r1, b_l1.reshape(1, D))
    part1, degp = _make_agg(True)(z1, srcf, dstf, srcs, dsts, zrow)
    p0, p1 = part1[:N], part1[N_PAD:N_PAD + N]
    dg = degp[:, :N].T
    z2, r2 = _mid(p0, p1, dg, r1, W_l2, W_r2, b_l2.reshape(1, D))
    part2 = _make_agg(False)(z2, srcf, dstf, srcs, dsts, zrow)
    q0, q1 = part2[:N], part2[N_PAD:N_PAD + N]
    return _final(q0, q1, dg, r2)
